# baseline (speedup 1.0000x reference)
import functools

import jax
import jax.numpy as jnp
from jax.experimental import pallas as pl


def _mlp2_body(x_ref, w1_ref, b1_ref, w2_ref, b2_ref, o_ref):
    x = x_ref[...]
    h = jnp.dot(x, w1_ref[...], preferred_element_type=jnp.float32)
    h = jnp.maximum(h + b1_ref[...], 0.0)
    o = jnp.dot(h, w2_ref[...], preferred_element_type=jnp.float32)
    o_ref[...] = o + b2_ref[...]


@functools.partial(jax.jit, static_argnames=("bm",))
def _mlp2(x, w1, b1, w2, b2, bm=4096):
    m, k = x.shape
    out = w2.shape[1]
    mp = ((m + bm - 1) // bm) * bm
    if mp != m:
        x = jnp.pad(x, ((0, mp - m), (0, 0)))
    grid = (mp // bm,)
    y = pl.pallas_call(
        _mlp2_body,
        grid=grid,
        in_specs=[
            pl.BlockSpec((bm, k), lambda i: (i, 0)),
            pl.BlockSpec((k, w1.shape[1]), lambda i: (0, 0)),
            pl.BlockSpec((1, w1.shape[1]), lambda i: (0, 0)),
            pl.BlockSpec((w1.shape[1], out), lambda i: (0, 0)),
            pl.BlockSpec((1, out), lambda i: (0, 0)),
        ],
        out_specs=pl.BlockSpec((bm, out), lambda i: (i, 0)),
        out_shape=jax.ShapeDtypeStruct((mp, out), jnp.float32),
    )(x, w1, b1[None, :], w2, b2[None, :])
    return y[:m]


def _pos_embed(t, embd_size=64, max_positions=200):
    half = embd_size // 2
    freqs = jnp.arange(half, dtype=jnp.float32) / float(half)
    freqs = (1.0 / max_positions) ** freqs
    x = t[:, None] * freqs[None, :]
    return jnp.concatenate([jnp.cos(x), jnp.sin(x)], axis=1)


def kernel(t, global_energy, features_0, energy_corrupted, layer, edge_index, graph_id, params):
    N = features_0.shape[0]

    time_embd = _pos_embed(t, 64, 200)
    node_features = jnp.concatenate(
        [features_0, jnp.take(params['layer_emb'], layer, axis=0)], axis=1)

    (pw1, pb1), (pw2, pb2) = params['particle']
    condition = _mlp2(global_energy, pw1, pb1, pw2, pb2, bm=64)

    gt = jnp.take(time_embd, graph_id, axis=0)
    gc = jnp.take(condition, graph_id, axis=0)

    (cw1, cb1), (cw2, cb2) = params['cell']
    cell_embedding = _mlp2(
        jnp.concatenate([energy_corrupted, node_features], axis=1),
        cw1, cb1, cw2, cb2)

    (iw1, ib1), (iw2, ib2) = params['init']
    h = _mlp2(jnp.concatenate([cell_embedding, gt, gc], axis=1),
              iw1, ib1, iw2, ib2)

    src = edge_index[0]
    dst = edge_index[1]
    for edge_p, node_p in params['mpnn']:
        (ew1, eb1), (ew2, eb2) = edge_p
        m = _mlp2(
            jnp.concatenate([jnp.take(h, src, axis=0),
                             jnp.take(h, dst, axis=0)], axis=1),
            ew1, eb1, ew2, eb2, bm=8192)
        agg = jax.ops.segment_sum(m, dst, num_segments=N)
        (nw1, nb1), (nw2, nb2) = node_p
        h = h + _mlp2(jnp.concatenate([h, agg], axis=1),
                      nw1, nb1, nw2, nb2)

    (fw1, fb1), (fw2, fb2) = params['noise']
    updated = jnp.concatenate([h, cell_embedding, gt, gc], axis=1)
    F_x = _mlp2(updated, fw1, fb1, fw2, fb2)
    return F_x
